# baseline (speedup 1.0000x reference)
import functools

import jax
import jax.numpy as jnp
from jax import lax
from jax.experimental import pallas as pl
from jax.experimental.pallas import tpu as pltpu
from jax.experimental.pallas import tpu_sc as plsc

B = 1024
L = 50
VOCAB = 100000
HIDDEN = 64

NC = 2
NS = 16
NW = NC * NS
H_PER_W = HIDDEN // NW
CHUNK = 256
NCHUNK = B // CHUNK
BG = CHUNK // 16


def _sc_body(words_hbm, w_hbm, b_hbm, out_hbm, wrow_v, wc0, wc1, outrow_v, bvec_v):
    wid = lax.axis_index("s") * NC + lax.axis_index("c")
    pltpu.sync_copy(b_hbm, bvec_v.at[pl.ds(0, HIDDEN)])
    lanes = lax.iota(jnp.int32, 16)
    wcs = [wc0, wc1]

    def scoped(sem0, sem1, semw):
        sems = [sem0, sem1]

        def words_copy(c):
            return pltpu.async_copy(
                words_hbm.at[pl.ds(c * (CHUNK * L), CHUNK * L)],
                wcs[c % 2],
                sems[c % 2],
            )

        def w_copy(h):
            return pltpu.async_copy(w_hbm.at[h], wrow_v, semw)

        wh = w_copy(wid * H_PER_W)
        for hi in range(H_PER_W):
            h = wid * H_PER_W + hi
            handles = [words_copy(0), words_copy(1)]
            wh.wait()
            bh = plsc.load_gather(bvec_v, [jnp.full((16,), 0, jnp.int32) + h])
            for c in range(NCHUNK):
                handles[c % 2].wait()
                wordsc = wcs[c % 2]

                def bg_body(g, _, bh=bh, wordsc=wordsc, c=c):
                    base = (g * 16 + lanes) * L
                    acc0 = bh
                    acc1 = jnp.zeros((16,), jnp.float32)
                    acc2 = jnp.zeros((16,), jnp.float32)
                    acc3 = jnp.zeros((16,), jnp.float32)
                    for j in range(0, L, 4):
                        w0 = plsc.load_gather(wordsc, [base + j])
                        acc0 = acc0 + plsc.load_gather(wrow_v, [w0])
                        w1 = plsc.load_gather(wordsc, [base + (j + 1)])
                        acc1 = acc1 + plsc.load_gather(wrow_v, [w1])
                        if j + 2 < L:
                            w2 = plsc.load_gather(wordsc, [base + (j + 2)])
                            acc2 = acc2 + plsc.load_gather(wrow_v, [w2])
                            w3 = plsc.load_gather(wordsc, [base + (j + 3)])
                            acc3 = acc3 + plsc.load_gather(wrow_v, [w3])
                    outrow_v[pl.ds(c * CHUNK + g * 16, 16)] = (
                        (acc0 + acc1) + (acc2 + acc3))
                    return 0

                lax.fori_loop(0, BG, bg_body, 0)
                if hi + 1 < H_PER_W and c == NCHUNK - 1:
                    wh = w_copy(h + 1)
                if c + 2 < NCHUNK:
                    handles[c % 2] = words_copy(c + 2)
            pltpu.sync_copy(outrow_v, out_hbm.at[h])

    pl.run_scoped(scoped, pltpu.SemaphoreType.DMA, pltpu.SemaphoreType.DMA,
                  pltpu.SemaphoreType.DMA)


@functools.partial(jax.jit, donate_argnums=())
def _launch(words_flat, W, b):
    mesh = plsc.VectorSubcoreMesh(core_axis_name="c", subcore_axis_name="s")
    f = pl.kernel(
        _sc_body,
        out_type=jax.ShapeDtypeStruct((HIDDEN, B), jnp.float32),
        mesh=mesh,
        scratch_types=[
            pltpu.VMEM((VOCAB,), jnp.float32),
            pltpu.VMEM((CHUNK * L,), jnp.int32),
            pltpu.VMEM((CHUNK * L,), jnp.int32),
            pltpu.VMEM((B,), jnp.float32),
            pltpu.VMEM((128,), jnp.float32),
        ],
        compiler_params=pltpu.CompilerParams(needs_layout_passes=False),
    )
    return f(words_flat, W, b)


def _tc_transpose_body(x_ref, o_ref):
    o_ref[...] = x_ref[...].T


@jax.jit
def _tc_transpose(x):
    return pl.pallas_call(
        _tc_transpose_body,
        grid=(B // 128,),
        in_specs=[pl.BlockSpec((HIDDEN, 128), lambda i: (0, i))],
        out_specs=pl.BlockSpec((128, HIDDEN), lambda i: (i, 0)),
        out_shape=jax.ShapeDtypeStruct((B, HIDDEN), jnp.float32),
    )(x)


def kernel(words, W, b):
    words_flat = words.reshape(-1).astype(jnp.int32)
    out_t = _launch(words_flat, W, b)
    return _tc_transpose(out_t)


# speedup vs baseline: 1.1246x; 1.1246x over previous
import functools

import jax
import jax.numpy as jnp
from jax import lax
from jax.experimental import pallas as pl
from jax.experimental.pallas import tpu as pltpu
from jax.experimental.pallas import tpu_sc as plsc

B = 1024
L = 50
VOCAB = 100000
HIDDEN = 64

NC = 2
NS = 16
NW = NC * NS
H_PER_W = HIDDEN // NW
CHUNK = 256
NCHUNK = B // CHUNK
BG = CHUNK // 16


def _sc_body(words_hbm, w_hbm, b_hbm, out_hbm, wrow_v, wc0, wc1, outrow_v, bvec_v):
    wid = lax.axis_index("s") * NC + lax.axis_index("c")
    pltpu.sync_copy(b_hbm, bvec_v.at[pl.ds(0, HIDDEN)])
    lanes = lax.iota(jnp.int32, 16)
    wcs = [wc0, wc1]

    def scoped(sem0, sem1, semw):
        sems = [sem0, sem1]

        def words_copy(c):
            return pltpu.async_copy(
                words_hbm.at[pl.ds(c * (CHUNK * L), CHUNK * L)],
                wcs[c % 2],
                sems[c % 2],
            )

        def w_copy(h):
            return pltpu.async_copy(w_hbm.at[h], wrow_v, semw)

        wh = w_copy(wid * H_PER_W)
        for hi in range(H_PER_W):
            h = wid * H_PER_W + hi
            handles = [words_copy(0), words_copy(1)]
            wh.wait()
            bh = plsc.load_gather(bvec_v, [jnp.full((16,), 0, jnp.int32) + h])
            for c in range(NCHUNK):
                handles[c % 2].wait()
                wordsc = wcs[c % 2]

                def bg_body(g, _, bh=bh, wordsc=wordsc, c=c):
                    base = (g * 16 + lanes) * L
                    acc0 = bh
                    acc1 = jnp.zeros((16,), jnp.float32)
                    acc2 = jnp.zeros((16,), jnp.float32)
                    acc3 = jnp.zeros((16,), jnp.float32)
                    for j in range(0, L, 4):
                        w0 = plsc.load_gather(wordsc, [base + j])
                        acc0 = acc0 + plsc.load_gather(wrow_v, [w0])
                        w1 = plsc.load_gather(wordsc, [base + (j + 1)])
                        acc1 = acc1 + plsc.load_gather(wrow_v, [w1])
                        if j + 2 < L:
                            w2 = plsc.load_gather(wordsc, [base + (j + 2)])
                            acc2 = acc2 + plsc.load_gather(wrow_v, [w2])
                            w3 = plsc.load_gather(wordsc, [base + (j + 3)])
                            acc3 = acc3 + plsc.load_gather(wrow_v, [w3])
                    outrow_v[pl.ds(c * CHUNK + g * 16, 16)] = (
                        (acc0 + acc1) + (acc2 + acc3))
                    return 0

                lax.fori_loop(0, BG, bg_body, 0)
                if hi + 1 < H_PER_W and c == NCHUNK - 1:
                    wh = w_copy(h + 1)
                if c + 2 < NCHUNK:
                    handles[c % 2] = words_copy(c + 2)
            pltpu.sync_copy(outrow_v, out_hbm.at[h])

    pl.run_scoped(scoped, pltpu.SemaphoreType.DMA, pltpu.SemaphoreType.DMA,
                  pltpu.SemaphoreType.DMA)


@functools.partial(jax.jit, donate_argnums=())
def _launch(words_flat, W, b):
    mesh = plsc.VectorSubcoreMesh(core_axis_name="c", subcore_axis_name="s")
    f = pl.kernel(
        _sc_body,
        out_type=jax.ShapeDtypeStruct((HIDDEN, B), jnp.float32),
        mesh=mesh,
        scratch_types=[
            pltpu.VMEM((VOCAB,), jnp.float32),
            pltpu.VMEM((CHUNK * L,), jnp.int32),
            pltpu.VMEM((CHUNK * L,), jnp.int32),
            pltpu.VMEM((B,), jnp.float32),
            pltpu.VMEM((128,), jnp.float32),
        ],
        compiler_params=pltpu.CompilerParams(needs_layout_passes=False),
    )
    return f(words_flat, W, b)


def kernel(words, W, b):
    words_flat = words.reshape(-1).astype(jnp.int32)
    out_t = _launch(words_flat, W, b)
    return out_t.T
